# Initial kernel scaffold; baseline (speedup 1.0000x reference)
import jax
import jax.numpy as jnp
from jax.experimental import pallas as pl


def kernel(features, Wc, bc, Wb, bb):
    raise NotImplementedError("write your pallas kernel here")


# speedup vs baseline: 1.3923x; 1.3923x over previous
import jax
import jax.numpy as jnp
from jax.experimental import pallas as pl

_NUM_LEVELS = 3
_NUM_CLASSES = 2
_NUM_ANCHORS = 10
_C = 512
_HW_BLK = 2048


def _head_kernel(x_ref, wc_ref, bc_ref, wb_ref, bb_ref, cls_ref, box_ref):
    x = x_ref[0, 0]
    wc = wc_ref[0]
    wb = wb_ref[0]
    dn = (((0,), (1,)), ((), ()))
    cls = jax.lax.dot_general(x, wc, dn, preferred_element_type=jnp.float32)
    box = jax.lax.dot_general(x, wb, dn, preferred_element_type=jnp.float32)
    cls_ref[0, 0] = cls + bc_ref[0]
    box_ref[0, 0] = box + bb_ref[0]


def kernel(features, Wc, bc, Wb, bb):
    L, B, C, H, W = features.shape
    HW = H * W
    Oc = Wc.shape[1]
    Ob = Wb.shape[1]
    f = features.reshape(L, B, C, HW)
    bc2 = bc.reshape(L, 1, Oc)
    bb2 = bb.reshape(L, 1, Ob)
    nblk = HW // _HW_BLK

    grid = (L, B, nblk)
    cls_out, box_out = pl.pallas_call(
        _head_kernel,
        grid=grid,
        in_specs=[
            pl.BlockSpec((1, 1, C, _HW_BLK), lambda l, b, h: (l, b, 0, h)),
            pl.BlockSpec((1, Oc, C), lambda l, b, h: (l, 0, 0)),
            pl.BlockSpec((1, 1, Oc), lambda l, b, h: (l, 0, 0)),
            pl.BlockSpec((1, Ob, C), lambda l, b, h: (l, 0, 0)),
            pl.BlockSpec((1, 1, Ob), lambda l, b, h: (l, 0, 0)),
        ],
        out_specs=[
            pl.BlockSpec((1, 1, _HW_BLK, Oc), lambda l, b, h: (b, l, h, 0)),
            pl.BlockSpec((1, 1, _HW_BLK, Ob), lambda l, b, h: (b, l, h, 0)),
        ],
        out_shape=[
            jax.ShapeDtypeStruct((B, L, HW, Oc), jnp.float32),
            jax.ShapeDtypeStruct((B, L, HW, Ob), jnp.float32),
        ],
    )(f, Wc, bc2, Wb, bb2)

    cls_score = cls_out.reshape(B, L * HW * _NUM_ANCHORS, _NUM_CLASSES)
    bbox_pred = box_out.reshape(B, L * HW * _NUM_ANCHORS, 4)
    return (cls_score, bbox_pred)


# speedup vs baseline: 2.3383x; 1.6794x over previous
import jax
import jax.numpy as jnp
from jax.experimental import pallas as pl

_NUM_CLASSES = 2
_NUM_ANCHORS = 10
_HB = 16


def _head_kernel(x_ref, wc_ref, bc_ref, wb_ref, bb_ref, cls_ref, box_ref):
    c = x_ref.shape[2]
    n = x_ref.shape[3] * x_ref.shape[4]
    x = x_ref[0, 0].reshape(c, n)
    dn = (((1,), (0,)), ((), ()))
    cls_ref[0] = jax.lax.dot_general(
        wc_ref[0], x, dn, preferred_element_type=jnp.float32) + bc_ref[0]
    box_ref[0] = jax.lax.dot_general(
        wb_ref[0], x, dn, preferred_element_type=jnp.float32) + bb_ref[0]


def kernel(features, Wc, bc, Wb, bb):
    L, B, C, H, W = features.shape
    Oc = Wc.shape[1]
    Ob = Wb.shape[1]
    A = _NUM_ANCHORS
    K = _NUM_CLASSES
    HW = H * W
    N = L * HW * A
    nblk = H // _HB
    NB = _HB * W

    Wcp = Wc.reshape(L, A, K, C).transpose(0, 2, 1, 3).reshape(L, Oc, C)
    bcp = bc.reshape(L, A, K).transpose(0, 2, 1).reshape(L, Oc, 1)
    Wbp = Wb.reshape(L, A, 4, C).transpose(0, 2, 1, 3).reshape(L, Ob, C)
    bbp = bb.reshape(L, A, 4).transpose(0, 2, 1).reshape(L, Ob, 1)

    grid = (L, B, nblk)
    cls_t, box_t = pl.pallas_call(
        _head_kernel,
        grid=grid,
        in_specs=[
            pl.BlockSpec((1, 1, C, _HB, W), lambda l, b, h: (l, b, 0, h, 0)),
            pl.BlockSpec((1, Oc, C), lambda l, b, h: (l, 0, 0)),
            pl.BlockSpec((1, Oc, 1), lambda l, b, h: (l, 0, 0)),
            pl.BlockSpec((1, Ob, C), lambda l, b, h: (l, 0, 0)),
            pl.BlockSpec((1, Ob, 1), lambda l, b, h: (l, 0, 0)),
        ],
        out_specs=[
            pl.BlockSpec((1, Oc, NB), lambda l, b, h: (b, 0, l * nblk + h)),
            pl.BlockSpec((1, Ob, NB), lambda l, b, h: (b, 0, l * nblk + h)),
        ],
        out_shape=[
            jax.ShapeDtypeStruct((B, Oc, L * HW), jnp.float32),
            jax.ShapeDtypeStruct((B, Ob, L * HW), jnp.float32),
        ],
    )(features, Wcp, bcp, Wbp, bbp)

    cls_score = (cls_t.reshape(B, K, A, L * HW)
                 .transpose(0, 3, 2, 1)
                 .reshape(B, N, K))
    bbox_pred = (box_t.reshape(B, 4, A, L * HW)
                 .transpose(0, 3, 2, 1)
                 .reshape(B, N, 4))
    return (cls_score, bbox_pred)
